# baseline (speedup 1.0000x reference)
import functools

import jax
import jax.numpy as jnp
from jax.experimental import pallas as pl
from jax.experimental.pallas import tpu as pltpu


def _expand_matrix(w, s):
    p = jnp.arange(w, dtype=jnp.int32)
    q = jnp.arange(s * s * w, dtype=jnp.int32)
    return ((q[None, :] % (s * w)) // s == p[:, None]).astype(jnp.float32)


def _up_body(r_ref, xr_ref, xi_ref, o_ref):
    p = pl.program_id(1)
    r = r_ref[...]

    @pl.when(p == 0)
    def _():
        o_ref[0] = jnp.dot(xr_ref[...], r, preferred_element_type=jnp.float32)

    @pl.when(p == 1)
    def _():
        o_ref[0] = jnp.dot(xi_ref[...], r, preferred_element_type=jnp.float32)


@functools.partial(jax.jit, static_argnames=())
def kernel(xr, xi):
    n, c, h, w = xr.shape
    s = 2
    t = n * c * h
    m = s * s * w

    br = 2048
    while t % br:
        br //= 2
    grid = t // br

    r = _expand_matrix(w, s)
    xr2 = xr.reshape(t, w)
    xi2 = xi.reshape(t, w)

    out = pl.pallas_call(
        _up_body,
        out_shape=jax.ShapeDtypeStruct((2, t, m), jnp.float32),
        grid=(grid, 2),
        in_specs=[
            pl.BlockSpec((w, m), lambda i, p: (0, 0)),
            pl.BlockSpec((br, w), lambda i, p: (i, 0)),
            pl.BlockSpec((br, w), lambda i, p: (i, 0)),
        ],
        out_specs=pl.BlockSpec((1, br, m), lambda i, p: (p, i, 0)),
        compiler_params=pltpu.CompilerParams(
            dimension_semantics=("arbitrary", "arbitrary")),
        cost_estimate=pl.CostEstimate(
            flops=2 * 2 * t * w * m,
            transcendentals=0,
            bytes_accessed=4 * (2 * t * w + 2 * t * m + w * m)),
    )(r, xr2, xi2)

    return out.reshape(2, n, c, h * s, s * w)


# speedup vs baseline: 1.0640x; 1.0640x over previous
import functools

import jax
import jax.numpy as jnp
from jax.experimental import pallas as pl
from jax.experimental.pallas import tpu as pltpu


def _expand_matrix(w, s):
    p = jnp.arange(w, dtype=jnp.int32)
    q = jnp.arange(s * s * w, dtype=jnp.int32)
    return ((q[None, :] % (s * w)) // s == p[:, None]).astype(jnp.float32)


def _up_body(r_ref, xr_ref, xi_ref, o_ref):
    r = r_ref[...]
    o_ref[0] = jnp.dot(xr_ref[...], r, preferred_element_type=jnp.float32)
    o_ref[1] = jnp.dot(xi_ref[...], r, preferred_element_type=jnp.float32)


@functools.partial(jax.jit, static_argnames=())
def kernel(xr, xi):
    n, c, h, w = xr.shape
    s = 2
    t = n * c * h
    m = s * s * w

    br = 4096
    while t % br:
        br //= 2
    grid = t // br

    r = _expand_matrix(w, s)
    xr2 = xr.reshape(t, w)
    xi2 = xi.reshape(t, w)

    out = pl.pallas_call(
        _up_body,
        out_shape=jax.ShapeDtypeStruct((2, t, m), jnp.float32),
        grid=(grid,),
        in_specs=[
            pl.BlockSpec((w, m), lambda i: (0, 0)),
            pl.BlockSpec((br, w), lambda i: (i, 0)),
            pl.BlockSpec((br, w), lambda i: (i, 0)),
        ],
        out_specs=pl.BlockSpec((2, br, m), lambda i: (0, i, 0)),
        compiler_params=pltpu.CompilerParams(
            dimension_semantics=("arbitrary",)),
        cost_estimate=pl.CostEstimate(
            flops=2 * 2 * t * w * m,
            transcendentals=0,
            bytes_accessed=4 * (2 * t * w + 2 * t * m + w * m)),
    )(r, xr2, xi2)

    return out.reshape(2, n, c, h * s, s * w)
